# Initial kernel scaffold; baseline (speedup 1.0000x reference)
import jax
import jax.numpy as jnp
from jax.experimental import pallas as pl


def kernel(J, b, mp_w1, mp_b1, mp_w2, mp_b2, mp_w3, mp_b3, gru_wih, gru_whh, gru_bih, gru_bhh, ro_w1, ro_b1, ro_w2, ro_b2, ro_w3, ro_b3):
    raise NotImplementedError("write your pallas kernel here")


# speedup vs baseline: 31.0862x; 31.0862x over previous
import jax
import jax.numpy as jnp
from jax.experimental import pallas as pl
from jax.experimental.pallas import tpu as pltpu

_N = 512
_STATE = 64
_MSG = 64
_HID = 128
_IN_MP = 2 * _STATE + 3
_NSTEPS = 4
_TI = 16
_E = _TI * _N

_bf16 = jnp.bfloat16


def _dot(a, b):
    return jax.lax.dot(a.astype(_bf16), b.astype(_bf16),
                       preferred_element_type=jnp.float32)


def _ggnn_kernel(
    j_ref, b_full_ref, b_tile_ref,
    w1t_ref, b1_ref, w2t_ref, b2_ref, w3t_ref, b3_ref,
    wih_r_ref, wih_z_ref, wih_n_ref,
    whh_r_ref, whh_z_ref, whh_n_ref,
    bih_r_ref, bih_z_ref, bih_n_ref,
    bhh_r_ref, bhh_z_ref, bhh_n_ref,
    ro_w1t_ref, ro_b1_ref, ro_w2t_ref, ro_b2_ref, ro_w3t_ref, ro_b3_ref,
    out_ref,
    h_scr, h_old,
):
    s = pl.program_id(0)
    i = pl.program_id(1)

    @pl.when(jnp.logical_and(s == 0, i == 0))
    def _init():
        h_scr[...] = jnp.zeros_like(h_scr)

    @pl.when(i == 0)
    def _snap():
        h_old[...] = h_scr[...]

    hi = h_old[pl.ds(i * _TI, _TI), :]
    h_all = h_old[...]

    jb = j_ref[...]
    j3 = jb[:, :, None]

    em3 = jnp.concatenate([
        jnp.broadcast_to(hi[:, None, :], (_TI, _N, _STATE)),
        jnp.broadcast_to(h_all[None, :, :], (_TI, _N, _STATE)),
        j3,
        jnp.broadcast_to(b_tile_ref[...][:, None, :], (_TI, _N, 1)),
        jnp.broadcast_to(b_full_ref[...][None, :, :], (_TI, _N, 1)),
    ], axis=2)
    em = em3.reshape(_E, _IN_MP)

    x1 = jnp.maximum(_dot(em, w1t_ref[...]) + b1_ref[...], 0.0)
    x2 = jnp.maximum(_dot(x1, w2t_ref[...]) + b2_ref[...], 0.0)
    x3 = _dot(x2, w3t_ref[...]) + b3_ref[...]
    mask = (j3 != 0.0).astype(jnp.float32)
    msg = jnp.sum(x3.reshape(_TI, _N, _MSG) * mask, axis=1)

    i_r = _dot(msg, wih_r_ref[...]) + bih_r_ref[...]
    i_z = _dot(msg, wih_z_ref[...]) + bih_z_ref[...]
    i_n = _dot(msg, wih_n_ref[...]) + bih_n_ref[...]
    h_r = _dot(hi, whh_r_ref[...]) + bhh_r_ref[...]
    h_z = _dot(hi, whh_z_ref[...]) + bhh_z_ref[...]
    h_n = _dot(hi, whh_n_ref[...]) + bhh_n_ref[...]
    r = jax.nn.sigmoid(i_r + h_r)
    z = jax.nn.sigmoid(i_z + h_z)
    n = jnp.tanh(i_n + r * h_n)
    h_new = (1.0 - z) * n + z * hi
    h_scr[pl.ds(i * _TI, _TI), :] = h_new

    @pl.when(s == _NSTEPS - 1)
    def _readout():
        r1 = jnp.maximum(_dot(h_new, ro_w1t_ref[...]) + ro_b1_ref[...], 0.0)
        r2 = jnp.maximum(_dot(r1, ro_w2t_ref[...]) + ro_b2_ref[...], 0.0)
        logits = _dot(r2, ro_w3t_ref[...]) + ro_b3_ref[...]
        sg = jax.nn.sigmoid(logits)
        out_ref[pl.ds(i * _TI, _TI), :] = sg / jnp.sum(sg, axis=1, keepdims=True)


def kernel(J, b, mp_w1, mp_b1, mp_w2, mp_b2, mp_w3, mp_b3, gru_wih, gru_whh,
           gru_bih, gru_bhh, ro_w1, ro_b1, ro_w2, ro_b2, ro_w3, ro_b3):
    f32 = jnp.float32
    J = J.astype(f32)
    b2d = b.reshape(_N, 1).astype(f32)
    w1t = mp_w1.T
    b1 = mp_b1.reshape(1, _HID)
    w2t = mp_w2.T
    b2 = mp_b2.reshape(1, _HID)
    w3t = mp_w3.T
    b3 = mp_b3.reshape(1, _MSG)
    wih_r = gru_wih[:_STATE].T
    wih_z = gru_wih[_STATE:2 * _STATE].T
    wih_n = gru_wih[2 * _STATE:].T
    whh_r = gru_whh[:_STATE].T
    whh_z = gru_whh[_STATE:2 * _STATE].T
    whh_n = gru_whh[2 * _STATE:].T
    bih_r = gru_bih[:_STATE].reshape(1, _STATE)
    bih_z = gru_bih[_STATE:2 * _STATE].reshape(1, _STATE)
    bih_n = gru_bih[2 * _STATE:].reshape(1, _STATE)
    bhh_r = gru_bhh[:_STATE].reshape(1, _STATE)
    bhh_z = gru_bhh[_STATE:2 * _STATE].reshape(1, _STATE)
    bhh_n = gru_bhh[2 * _STATE:].reshape(1, _STATE)
    ro_w1t = ro_w1.T
    rb1 = ro_b1.reshape(1, -1)
    ro_w2t = ro_w2.T
    rb2 = ro_b2.reshape(1, -1)
    ro_w3t = ro_w3.T
    rb3 = ro_b3.reshape(1, -1)

    whole = pl.BlockSpec(index_map=lambda s, i: (0, 0))
    grid = (_NSTEPS, _N // _TI)
    out = pl.pallas_call(
        _ggnn_kernel,
        grid=grid,
        in_specs=[
            pl.BlockSpec((_TI, _N), lambda s, i: (i, 0)),
            pl.BlockSpec((_N, 1), lambda s, i: (0, 0)),
            pl.BlockSpec((_TI, 1), lambda s, i: (i, 0)),
        ] + [whole] * 24,
        out_specs=pl.BlockSpec((_N, 2), lambda s, i: (0, 0)),
        out_shape=jax.ShapeDtypeStruct((_N, 2), f32),
        scratch_shapes=[
            pltpu.VMEM((_N, _STATE), f32),
            pltpu.VMEM((_N, _STATE), f32),
        ],
        compiler_params=pltpu.CompilerParams(
            dimension_semantics=("arbitrary", "arbitrary"),
        ),
    )(
        J, b2d, b2d,
        w1t, b1, w2t, b2, w3t, b3,
        wih_r, wih_z, wih_n,
        whh_r, whh_z, whh_n,
        bih_r, bih_z, bih_n,
        bhh_r, bhh_z, bhh_n,
        ro_w1t, rb1, ro_w2t, rb2, ro_w3t, rb3,
    )
    return out


# speedup vs baseline: 33.8902x; 1.0902x over previous
import jax
import jax.numpy as jnp
from jax.experimental import pallas as pl
from jax.experimental.pallas import tpu as pltpu

_N = 512
_STATE = 64
_MSG = 64
_HID = 128
_IN_MP = 2 * _STATE + 3
_NSTEPS = 4
_TI = 32
_E = _TI * _N

_bf16 = jnp.bfloat16


def _dot(a, b):
    return jax.lax.dot(a.astype(_bf16), b.astype(_bf16),
                       preferred_element_type=jnp.float32)


def _ggnn_kernel(
    j_ref, b_full_ref, b_tile_ref,
    w1t_ref, b1_ref, w2t_ref, b2_ref, w3t_ref, b3_ref,
    wih_r_ref, wih_z_ref, wih_n_ref,
    whh_r_ref, whh_z_ref, whh_n_ref,
    bih_r_ref, bih_z_ref, bih_n_ref,
    bhh_r_ref, bhh_z_ref, bhh_n_ref,
    ro_w1t_ref, ro_b1_ref, ro_w2t_ref, ro_b2_ref, ro_w3t_ref, ro_b3_ref,
    out_ref,
    h_scr, h_old,
):
    s = pl.program_id(0)
    i = pl.program_id(1)

    @pl.when(jnp.logical_and(s == 0, i == 0))
    def _init():
        h_scr[...] = jnp.zeros_like(h_scr)

    @pl.when(i == 0)
    def _snap():
        h_old[...] = h_scr[...]

    hi = h_old[pl.ds(i * _TI, _TI), :]
    h_all = h_old[...]

    jb = j_ref[...]
    j3 = jb[:, :, None]

    em3 = jnp.concatenate([
        jnp.broadcast_to(hi.astype(_bf16)[:, None, :], (_TI, _N, _STATE)),
        jnp.broadcast_to(h_all.astype(_bf16)[None, :, :], (_TI, _N, _STATE)),
        j3.astype(_bf16),
        jnp.broadcast_to(b_tile_ref[...].astype(_bf16)[:, None, :], (_TI, _N, 1)),
        jnp.broadcast_to(b_full_ref[...].astype(_bf16)[None, :, :], (_TI, _N, 1)),
    ], axis=2)
    em = em3.reshape(_E, _IN_MP)

    x1 = jnp.maximum(_dot(em, w1t_ref[...]) + b1_ref[...], 0.0)
    x2 = jnp.maximum(_dot(x1, w2t_ref[...]) + b2_ref[...], 0.0)
    x3 = _dot(x2, w3t_ref[...]) + b3_ref[...]
    mask = (j3 != 0.0).astype(jnp.float32)
    msg = jnp.sum(x3.reshape(_TI, _N, _MSG) * mask, axis=1)

    i_r = _dot(msg, wih_r_ref[...]) + bih_r_ref[...]
    i_z = _dot(msg, wih_z_ref[...]) + bih_z_ref[...]
    i_n = _dot(msg, wih_n_ref[...]) + bih_n_ref[...]
    h_r = _dot(hi, whh_r_ref[...]) + bhh_r_ref[...]
    h_z = _dot(hi, whh_z_ref[...]) + bhh_z_ref[...]
    h_n = _dot(hi, whh_n_ref[...]) + bhh_n_ref[...]
    r = jax.nn.sigmoid(i_r + h_r)
    z = jax.nn.sigmoid(i_z + h_z)
    n = jnp.tanh(i_n + r * h_n)
    h_new = (1.0 - z) * n + z * hi
    h_scr[pl.ds(i * _TI, _TI), :] = h_new

    @pl.when(s == _NSTEPS - 1)
    def _readout():
        r1 = jnp.maximum(_dot(h_new, ro_w1t_ref[...]) + ro_b1_ref[...], 0.0)
        r2 = jnp.maximum(_dot(r1, ro_w2t_ref[...]) + ro_b2_ref[...], 0.0)
        logits = _dot(r2, ro_w3t_ref[...]) + ro_b3_ref[...]
        sg = jax.nn.sigmoid(logits)
        out_ref[pl.ds(i * _TI, _TI), :] = sg / jnp.sum(sg, axis=1, keepdims=True)


def kernel(J, b, mp_w1, mp_b1, mp_w2, mp_b2, mp_w3, mp_b3, gru_wih, gru_whh,
           gru_bih, gru_bhh, ro_w1, ro_b1, ro_w2, ro_b2, ro_w3, ro_b3):
    f32 = jnp.float32
    J = J.astype(f32)
    b2d = b.reshape(_N, 1).astype(f32)
    w1t = mp_w1.T
    b1 = mp_b1.reshape(1, _HID)
    w2t = mp_w2.T
    b2 = mp_b2.reshape(1, _HID)
    w3t = mp_w3.T
    b3 = mp_b3.reshape(1, _MSG)
    wih_r = gru_wih[:_STATE].T
    wih_z = gru_wih[_STATE:2 * _STATE].T
    wih_n = gru_wih[2 * _STATE:].T
    whh_r = gru_whh[:_STATE].T
    whh_z = gru_whh[_STATE:2 * _STATE].T
    whh_n = gru_whh[2 * _STATE:].T
    bih_r = gru_bih[:_STATE].reshape(1, _STATE)
    bih_z = gru_bih[_STATE:2 * _STATE].reshape(1, _STATE)
    bih_n = gru_bih[2 * _STATE:].reshape(1, _STATE)
    bhh_r = gru_bhh[:_STATE].reshape(1, _STATE)
    bhh_z = gru_bhh[_STATE:2 * _STATE].reshape(1, _STATE)
    bhh_n = gru_bhh[2 * _STATE:].reshape(1, _STATE)
    ro_w1t = ro_w1.T
    rb1 = ro_b1.reshape(1, -1)
    ro_w2t = ro_w2.T
    rb2 = ro_b2.reshape(1, -1)
    ro_w3t = ro_w3.T
    rb3 = ro_b3.reshape(1, -1)

    whole = pl.BlockSpec(index_map=lambda s, i: (0, 0))
    grid = (_NSTEPS, _N // _TI)
    out = pl.pallas_call(
        _ggnn_kernel,
        grid=grid,
        in_specs=[
            pl.BlockSpec((_TI, _N), lambda s, i: (i, 0)),
            pl.BlockSpec((_N, 1), lambda s, i: (0, 0)),
            pl.BlockSpec((_TI, 1), lambda s, i: (i, 0)),
        ] + [whole] * 24,
        out_specs=pl.BlockSpec((_N, 2), lambda s, i: (0, 0)),
        out_shape=jax.ShapeDtypeStruct((_N, 2), f32),
        scratch_shapes=[
            pltpu.VMEM((_N, _STATE), f32),
            pltpu.VMEM((_N, _STATE), f32),
        ],
        compiler_params=pltpu.CompilerParams(
            dimension_semantics=("arbitrary", "arbitrary"),
        ),
    )(
        J, b2d, b2d,
        w1t, b1, w2t, b2, w3t, b3,
        wih_r, wih_z, wih_n,
        whh_r, whh_z, whh_n,
        bih_r, bih_z, bih_n,
        bhh_r, bhh_z, bhh_n,
        ro_w1t, rb1, ro_w2t, rb2, ro_w3t, rb3,
    )
    return out


# speedup vs baseline: 35.5045x; 1.0476x over previous
import jax
import jax.numpy as jnp
from jax.experimental import pallas as pl
from jax.experimental.pallas import tpu as pltpu

_N = 512
_STATE = 64
_MSG = 64
_HID = 128
_IN_MP = 2 * _STATE + 3
_NSTEPS = 4
_TI = 64
_E = _TI * _N

_bf16 = jnp.bfloat16


def _dot(a, b):
    return jax.lax.dot(a.astype(_bf16), b.astype(_bf16),
                       preferred_element_type=jnp.float32)


def _ggnn_kernel(
    j_ref, b_full_ref, b_tile_ref,
    w1t_ref, b1_ref, w2t_ref, b2_ref, w3t_ref, b3_ref,
    wih_r_ref, wih_z_ref, wih_n_ref,
    whh_r_ref, whh_z_ref, whh_n_ref,
    bih_r_ref, bih_z_ref, bih_n_ref,
    bhh_r_ref, bhh_z_ref, bhh_n_ref,
    ro_w1t_ref, ro_b1_ref, ro_w2t_ref, ro_b2_ref, ro_w3t_ref, ro_b3_ref,
    out_ref,
    h_scr, h_old,
):
    s = pl.program_id(0)
    i = pl.program_id(1)

    @pl.when(jnp.logical_and(s == 0, i == 0))
    def _init():
        h_scr[...] = jnp.zeros_like(h_scr)

    @pl.when(i == 0)
    def _snap():
        h_old[...] = h_scr[...]

    hi = h_old[pl.ds(i * _TI, _TI), :]
    h_all = h_old[...]

    jb = j_ref[...]
    j3 = jb[:, :, None]

    em3 = jnp.concatenate([
        jnp.broadcast_to(hi.astype(_bf16)[:, None, :], (_TI, _N, _STATE)),
        jnp.broadcast_to(h_all.astype(_bf16)[None, :, :], (_TI, _N, _STATE)),
        j3.astype(_bf16),
        jnp.broadcast_to(b_tile_ref[...].astype(_bf16)[:, None, :], (_TI, _N, 1)),
        jnp.broadcast_to(b_full_ref[...].astype(_bf16)[None, :, :], (_TI, _N, 1)),
    ], axis=2)
    em = em3.reshape(_E, _IN_MP)

    x1 = jnp.maximum(_dot(em, w1t_ref[...]) + b1_ref[...], 0.0)
    x2 = jnp.maximum(_dot(x1, w2t_ref[...]) + b2_ref[...], 0.0)
    x3 = _dot(x2, w3t_ref[...]) + b3_ref[...]
    mask = (j3 != 0.0).astype(jnp.float32)
    msg = jnp.sum(x3.reshape(_TI, _N, _MSG) * mask, axis=1)

    i_r = _dot(msg, wih_r_ref[...]) + bih_r_ref[...]
    i_z = _dot(msg, wih_z_ref[...]) + bih_z_ref[...]
    i_n = _dot(msg, wih_n_ref[...]) + bih_n_ref[...]
    h_r = _dot(hi, whh_r_ref[...]) + bhh_r_ref[...]
    h_z = _dot(hi, whh_z_ref[...]) + bhh_z_ref[...]
    h_n = _dot(hi, whh_n_ref[...]) + bhh_n_ref[...]
    r = jax.nn.sigmoid(i_r + h_r)
    z = jax.nn.sigmoid(i_z + h_z)
    n = jnp.tanh(i_n + r * h_n)
    h_new = (1.0 - z) * n + z * hi
    h_scr[pl.ds(i * _TI, _TI), :] = h_new

    @pl.when(s == _NSTEPS - 1)
    def _readout():
        r1 = jnp.maximum(_dot(h_new, ro_w1t_ref[...]) + ro_b1_ref[...], 0.0)
        r2 = jnp.maximum(_dot(r1, ro_w2t_ref[...]) + ro_b2_ref[...], 0.0)
        logits = _dot(r2, ro_w3t_ref[...]) + ro_b3_ref[...]
        sg = jax.nn.sigmoid(logits)
        out_ref[pl.ds(i * _TI, _TI), :] = sg / jnp.sum(sg, axis=1, keepdims=True)


def kernel(J, b, mp_w1, mp_b1, mp_w2, mp_b2, mp_w3, mp_b3, gru_wih, gru_whh,
           gru_bih, gru_bhh, ro_w1, ro_b1, ro_w2, ro_b2, ro_w3, ro_b3):
    f32 = jnp.float32
    J = J.astype(f32)
    b2d = b.reshape(_N, 1).astype(f32)
    w1t = mp_w1.T
    b1 = mp_b1.reshape(1, _HID)
    w2t = mp_w2.T
    b2 = mp_b2.reshape(1, _HID)
    w3t = mp_w3.T
    b3 = mp_b3.reshape(1, _MSG)
    wih_r = gru_wih[:_STATE].T
    wih_z = gru_wih[_STATE:2 * _STATE].T
    wih_n = gru_wih[2 * _STATE:].T
    whh_r = gru_whh[:_STATE].T
    whh_z = gru_whh[_STATE:2 * _STATE].T
    whh_n = gru_whh[2 * _STATE:].T
    bih_r = gru_bih[:_STATE].reshape(1, _STATE)
    bih_z = gru_bih[_STATE:2 * _STATE].reshape(1, _STATE)
    bih_n = gru_bih[2 * _STATE:].reshape(1, _STATE)
    bhh_r = gru_bhh[:_STATE].reshape(1, _STATE)
    bhh_z = gru_bhh[_STATE:2 * _STATE].reshape(1, _STATE)
    bhh_n = gru_bhh[2 * _STATE:].reshape(1, _STATE)
    ro_w1t = ro_w1.T
    rb1 = ro_b1.reshape(1, -1)
    ro_w2t = ro_w2.T
    rb2 = ro_b2.reshape(1, -1)
    ro_w3t = ro_w3.T
    rb3 = ro_b3.reshape(1, -1)

    whole = pl.BlockSpec(index_map=lambda s, i: (0, 0))
    grid = (_NSTEPS, _N // _TI)
    out = pl.pallas_call(
        _ggnn_kernel,
        grid=grid,
        in_specs=[
            pl.BlockSpec((_TI, _N), lambda s, i: (i, 0)),
            pl.BlockSpec((_N, 1), lambda s, i: (0, 0)),
            pl.BlockSpec((_TI, 1), lambda s, i: (i, 0)),
        ] + [whole] * 24,
        out_specs=pl.BlockSpec((_N, 2), lambda s, i: (0, 0)),
        out_shape=jax.ShapeDtypeStruct((_N, 2), f32),
        scratch_shapes=[
            pltpu.VMEM((_N, _STATE), f32),
            pltpu.VMEM((_N, _STATE), f32),
        ],
        compiler_params=pltpu.CompilerParams(
            dimension_semantics=("arbitrary", "arbitrary"),
        ),
    )(
        J, b2d, b2d,
        w1t, b1, w2t, b2, w3t, b3,
        wih_r, wih_z, wih_n,
        whh_r, whh_z, whh_n,
        bih_r, bih_z, bih_n,
        bhh_r, bhh_z, bhh_n,
        ro_w1t, rb1, ro_w2t, rb2, ro_w3t, rb3,
    )
    return out
